# baseline (speedup 1.0000x reference)
import jax
import jax.numpy as jnp
from jax.experimental import pallas as pl
from jax.experimental.pallas import tpu as pltpu

D_MODEL = 4096
NUM_EXPERTS = 64
TOP_K = 8
BLOCK_T = 1024
HALF_D = D_MODEL // 2


def _gate_body_e(xa_ref, xb_ref, wt_ref, bt_ref, vals_ref, idx_ref):
    dn = (((1,), (1,)), ((), ()))
    logits_t = jax.lax.dot_general(
        wt_ref[:, :HALF_D], xa_ref[...], dimension_numbers=dn,
        preferred_element_type=jnp.float32,
    )
    logits_t = logits_t + jax.lax.dot_general(
        wt_ref[:, HALF_D:], xb_ref[...], dimension_numbers=dn,
        preferred_element_type=jnp.float32,
    )
    logits_t = logits_t + bt_ref[...]
    m = jnp.max(logits_t, axis=0, keepdims=True)
    e = jnp.exp(logits_t - m)
    probs = e / jnp.sum(e, axis=0, keepdims=True)

    iota = jax.lax.broadcasted_iota(jnp.int32, probs.shape, 0)
    vals = []
    idxs = []
    work = probs
    for _ in range(TOP_K):
        mx = jnp.max(work, axis=0, keepdims=True)
        sel = jnp.min(jnp.where(work == mx, iota, NUM_EXPERTS), axis=0,
                      keepdims=True)
        vals.append(mx)
        idxs.append(sel)
        work = jnp.where(iota == sel, -jnp.inf, work)
    vals_ref[...] = jnp.concatenate(vals, axis=0)
    idx_ref[...] = jnp.concatenate(idxs, axis=0)


@jax.jit
def kernel(x, W_gate, b_gate):
    n_tokens = x.shape[0]
    grid = (n_tokens // BLOCK_T,)
    wt = W_gate.T
    bt = b_gate.reshape(NUM_EXPERTS, 1)
    vals_t, idx_t = pl.pallas_call(
        _gate_body_e,
        grid=grid,
        in_specs=[
            pl.BlockSpec((BLOCK_T, HALF_D), lambda i: (i, 0)),
            pl.BlockSpec((BLOCK_T, HALF_D), lambda i: (i, 1)),
            pl.BlockSpec((NUM_EXPERTS, D_MODEL), lambda i: (0, 0)),
            pl.BlockSpec((NUM_EXPERTS, 1), lambda i: (0, 0)),
        ],
        out_specs=[
            pl.BlockSpec((TOP_K, BLOCK_T), lambda i: (0, i)),
            pl.BlockSpec((TOP_K, BLOCK_T), lambda i: (0, i)),
        ],
        out_shape=[
            jax.ShapeDtypeStruct((TOP_K, n_tokens), jnp.float32),
            jax.ShapeDtypeStruct((TOP_K, n_tokens), jnp.int32),
        ],
        compiler_params=pltpu.CompilerParams(
            dimension_semantics=("parallel",),
        ),
    )(x, x, wt, bt)
    return vals_t.T, idx_t.T
